# baseline (speedup 1.0000x reference)
import functools

import jax
import jax.numpy as jnp
from jax import lax
from jax.experimental import pallas as pl
from jax.experimental.pallas import tpu as pltpu
from jax.experimental.pallas import tpu_sc as plsc

_EMBED_DIMS = [(33, 17), (2, 1), (8, 4), (3, 2), (3, 2), (3, 2), (3, 2), (3, 2), (20, 10)]
_COFFS = []
_c = 0
for _vv, _dd in _EMBED_DIMS:
    _COFFS.append(_c)
    _c += _dd
_TOTC = _c
_CPAD = 48
_NT = len(_EMBED_DIMS)

_NC, _NS = 2, 16
_NW = _NC * _NS


def _sc_body(bpw, x_hbm, e0, e1, e2, e3, e4, e5, e6, e7, e8, out_hbm,
             xt, t0, t1, t2, t3, t4, t5, t6, t7, t8, accT, sem_in, sem_out):
    tbls = [t0, t1, t2, t3, t4, t5, t6, t7, t8]
    wid = lax.axis_index("s") * _NC + lax.axis_index("c")
    base = wid * bpw
    loads = [pltpu.async_copy(x_hbm.at[pl.ds(base, bpw), :], xt, sem_in)]
    for src, dst in zip([e0, e1, e2, e3, e4, e5, e6, e7, e8], tbls):
        loads.append(pltpu.async_copy(src, dst, sem_in))
    for cp in loads:
        cp.wait()

    zeros16 = jnp.zeros((16,), jnp.float32)
    for j in range(_TOTC, _CPAD):
        for g in range(bpw // 16):
            accT[j, pl.ds(g * 16, 16)] = zeros16

    iota16 = lax.iota(jnp.int32, 16)

    @plsc.parallel_loop(0, bpw // 16, unroll=4)
    def _loop(g):
        rows = g * 16 + iota16
        for i in range(_NT):
            cf = plsc.load_gather(xt, [rows, jnp.full((16,), i + 1, jnp.int32)])
            cv = cf.astype(jnp.int32)
            for r in range(_EMBED_DIMS[i][1]):
                vals = plsc.load_gather(tbls[i], [cv, jnp.full((16,), r, jnp.int32)])
                accT[_COFFS[i] + r, pl.ds(g * 16, 16)] = vals

    pltpu.async_copy(accT, out_hbm.at[:, pl.ds(base, bpw)], sem_out).wait()


def _tc_body(nb, d_out, et_ref, xrow_ref, xr_ref, w_ref, b_ref, g_ref, be_ref, o_ref):
    xr = xr_ref[...]
    mean = jnp.sum(xr) * (1.0 / nb)
    var = jnp.sum((xr - mean) ** 2) * (1.0 / nb)
    a = g_ref[0, 0] * jax.lax.rsqrt(var + 1e-5)
    c = be_ref[0, 0] - mean * a

    cnT = xrow_ref[...] * a + c
    ih = lax.broadcasted_iota(jnp.int32, et_ref.shape, 0)
    eh = jnp.where(ih == _TOTC, cnT, et_ref[...])
    wfull = jnp.concatenate(
        [w_ref[...], jnp.zeros((d_out, _CPAD - _TOTC - 1), jnp.float32)], axis=1)
    z = jax.lax.dot_general(
        eh, wfull, (((0,), (1,)), ((), ())),
        preferred_element_type=jnp.float32, precision=jax.lax.Precision.HIGHEST)
    z = z + b_ref[...]
    z = z - jnp.max(z, axis=1, keepdims=True)
    ez = jnp.exp(z)
    o_ref[...] = ez / jnp.sum(ez, axis=1, keepdims=True)


def kernel(x, emb0, emb1, emb2, emb3, emb4, emb5, emb6, emb7, emb8, W, b, gamma, beta):
    tables = [emb0, emb1, emb2, emb3, emb4, emb5, emb6, emb7, emb8]
    B = x.shape[0]
    d_out = W.shape[0]
    bpw = B // _NW

    mesh = plsc.VectorSubcoreMesh(core_axis_name="c", subcore_axis_name="s")
    et = pl.kernel(
        functools.partial(_sc_body, bpw),
        out_type=jax.ShapeDtypeStruct((_CPAD, B), jnp.float32),
        mesh=mesh,
        scratch_types=[
            pltpu.VMEM((bpw, x.shape[1]), jnp.float32),
        ] + [pltpu.VMEM(t.shape, jnp.float32) for t in tables] + [
            pltpu.VMEM((_CPAD, bpw), jnp.float32),
            pltpu.SemaphoreType.DMA,
            pltpu.SemaphoreType.DMA,
        ],
        compiler_params=pltpu.CompilerParams(needs_layout_passes=False),
    )(x, *tables)

    xc = x[:, 0]
    xr = xc.reshape(128, B // 128)
    xrow = xc.reshape(1, B)
    b2 = b.reshape(1, d_out)
    g2 = gamma.reshape(1, 1)
    be2 = beta.reshape(1, 1)

    bb = 1024
    out = pl.pallas_call(
        functools.partial(_tc_body, float(B), d_out),
        grid=(B // bb,),
        in_specs=[
            pl.BlockSpec((_CPAD, bb), lambda i: (0, i)),
            pl.BlockSpec((1, bb), lambda i: (0, i)),
            pl.BlockSpec(xr.shape, lambda i: (0, 0)),
            pl.BlockSpec(W.shape, lambda i: (0, 0)),
            pl.BlockSpec((1, d_out), lambda i: (0, 0)),
            pl.BlockSpec((1, 1), lambda i: (0, 0)),
            pl.BlockSpec((1, 1), lambda i: (0, 0)),
        ],
        out_specs=pl.BlockSpec((bb, d_out), lambda i: (i, 0)),
        out_shape=jax.ShapeDtypeStruct((B, d_out), jnp.float32),
        compiler_params=pltpu.CompilerParams(fuse_transposed_lhs_in_matmul=True),
    )(et, xrow, xr, W, b2, g2, be2)
    return out


# speedup vs baseline: 1.8126x; 1.8126x over previous
import functools

import jax
import jax.numpy as jnp
from jax import lax
from jax.experimental import pallas as pl
from jax.experimental.pallas import tpu as pltpu
from jax.experimental.pallas import tpu_sc as plsc

_EMBED_DIMS = [(33, 17), (2, 1), (8, 4), (3, 2), (3, 2), (3, 2), (3, 2), (3, 2), (20, 10)]
_COFFS = []
_FBASE = []
_c = 0
_f = 0
for _vv, _dd in _EMBED_DIMS:
    _COFFS.append(_c)
    _FBASE.append(_f)
    _c += _dd
    _f += _vv * _dd
_TOTC = _c
_FLAT = _f
_FPAD = 832
_CPAD = 48
_NT = len(_EMBED_DIMS)

_NC, _NS = 2, 16
_NW = _NC * _NS


def _sc_body(bpw, nb, tflat_hbm, cat_hbm, out_hbm, catv, tflat, accT, sem_in, sem_out):
    wid = lax.axis_index("s") * _NC + lax.axis_index("c")
    base = wid * bpw
    loads = [pltpu.async_copy(tflat_hbm, tflat, sem_in)]
    for i in range(_NT):
        loads.append(pltpu.async_copy(cat_hbm.at[pl.ds(i * nb + base, bpw)],
                                      catv.at[pl.ds(i * bpw, bpw)], sem_in))
    for cp in loads:
        cp.wait()

    @plsc.parallel_loop(0, bpw // 16, unroll=2)
    def _loop(g):
        for i in range(_NT):
            d = _EMBED_DIMS[i][1]
            cv = catv[pl.ds(i * bpw + g * 16, 16)]
            fi = cv * d + _FBASE[i]
            for r in range(d):
                vals = plsc.load_gather(tflat, [fi + r])
                accT[_COFFS[i] + r, pl.ds(g * 16, 16)] = vals

    pltpu.async_copy(accT, out_hbm.at[:, pl.ds(base, bpw)], sem_out).wait()


def _tc_body(nb, d_out, et_ref, xrow_ref, xr_ref, w_ref, b_ref, g_ref, be_ref, o_ref):
    xr = xr_ref[...]
    mean = jnp.sum(xr) * (1.0 / nb)
    var = jnp.sum((xr - mean) ** 2) * (1.0 / nb)
    a = g_ref[0, 0] * jax.lax.rsqrt(var + 1e-5)
    c = be_ref[0, 0] - mean * a

    cnT = xrow_ref[...] * a + c
    ih = lax.broadcasted_iota(jnp.int32, et_ref.shape, 0)
    eh = jnp.where(ih == _TOTC, cnT,
                   jnp.where(ih > _TOTC, 0.0, et_ref[...]))
    wfull = jnp.concatenate(
        [w_ref[...], jnp.zeros((d_out, _CPAD - _TOTC - 1), jnp.float32)], axis=1)
    z = jax.lax.dot_general(
        eh, wfull, (((0,), (1,)), ((), ())),
        preferred_element_type=jnp.float32, precision=jax.lax.Precision.DEFAULT)
    z = z + b_ref[...]
    z = z - jnp.max(z, axis=1, keepdims=True)
    ez = jnp.exp(z)
    o_ref[...] = ez / jnp.sum(ez, axis=1, keepdims=True)


def kernel(x, emb0, emb1, emb2, emb3, emb4, emb5, emb6, emb7, emb8, W, b, gamma, beta):
    tables = [emb0, emb1, emb2, emb3, emb4, emb5, emb6, emb7, emb8]
    B = x.shape[0]
    d_out = W.shape[0]
    bpw = B // _NW

    tflat = jnp.concatenate([t.reshape(-1) for t in tables]
                            + [jnp.zeros((_FPAD - _FLAT,), jnp.float32)])
    cat_t = x[:, 1:].astype(jnp.int32).T.reshape(-1)

    mesh = plsc.VectorSubcoreMesh(core_axis_name="c", subcore_axis_name="s")
    et = pl.kernel(
        functools.partial(_sc_body, bpw, B),
        out_type=jax.ShapeDtypeStruct((_CPAD, B), jnp.float32),
        mesh=mesh,
        scratch_types=[
            pltpu.VMEM((_NT * bpw,), jnp.int32),
            pltpu.VMEM((_FPAD,), jnp.float32),
            pltpu.VMEM((_CPAD, bpw), jnp.float32),
            pltpu.SemaphoreType.DMA,
            pltpu.SemaphoreType.DMA,
        ],
        compiler_params=pltpu.CompilerParams(needs_layout_passes=False),
    )(tflat, cat_t)

    xc = x[:, 0]
    xr = xc.reshape(128, B // 128)
    xrow = xc.reshape(1, B)
    b2 = b.reshape(1, d_out)
    g2 = gamma.reshape(1, 1)
    be2 = beta.reshape(1, 1)

    bb = 8192
    out = pl.pallas_call(
        functools.partial(_tc_body, float(B), d_out),
        grid=(B // bb,),
        in_specs=[
            pl.BlockSpec((_CPAD, bb), lambda i: (0, i)),
            pl.BlockSpec((1, bb), lambda i: (0, i)),
            pl.BlockSpec(xr.shape, lambda i: (0, 0)),
            pl.BlockSpec(W.shape, lambda i: (0, 0)),
            pl.BlockSpec((1, d_out), lambda i: (0, 0)),
            pl.BlockSpec((1, 1), lambda i: (0, 0)),
            pl.BlockSpec((1, 1), lambda i: (0, 0)),
        ],
        out_specs=pl.BlockSpec((bb, d_out), lambda i: (i, 0)),
        out_shape=jax.ShapeDtypeStruct((B, d_out), jnp.float32),
        compiler_params=pltpu.CompilerParams(fuse_transposed_lhs_in_matmul=True),
    )(et, xrow, xr, W, b2, g2, be2)
    return out


# speedup vs baseline: 1.8377x; 1.0138x over previous
import functools

import jax
import jax.numpy as jnp
from jax import lax
from jax.experimental import pallas as pl
from jax.experimental.pallas import tpu as pltpu
from jax.experimental.pallas import tpu_sc as plsc

_EMBED_DIMS = [(33, 17), (2, 1), (8, 4), (3, 2), (3, 2), (3, 2), (3, 2), (3, 2), (20, 10)]
_COFFS = []
_FBASE = []
_c = 0
_f = 0
for _vv, _dd in _EMBED_DIMS:
    _COFFS.append(_c)
    _FBASE.append(_f)
    _c += _dd
    _f += _vv * _dd
_TOTC = _c
_FLAT = _f
_FPAD = 832
_CPAD = 48
_NT = len(_EMBED_DIMS)

_NC, _NS = 2, 16
_NW = _NC * _NS


def _sc_body(bpw, nb, tflat_hbm, cat_hbm, out_hbm, catv, tflat, accT, sem_in, sem_out):
    wid = lax.axis_index("s") * _NC + lax.axis_index("c")
    base = wid * bpw
    loads = [pltpu.async_copy(tflat_hbm, tflat, sem_in)]
    for i in range(_NT):
        loads.append(pltpu.async_copy(cat_hbm.at[pl.ds(i * nb + base, bpw)],
                                      catv.at[pl.ds(i * bpw, bpw)], sem_in))
    for cp in loads:
        cp.wait()

    @plsc.parallel_loop(0, bpw // 16, unroll=2)
    def _loop(g):
        for i in range(_NT):
            d = _EMBED_DIMS[i][1]
            cv = catv[pl.ds(i * bpw + g * 16, 16)]
            fi = cv * d + _FBASE[i]
            for r in range(d):
                vals = plsc.load_gather(tflat, [fi + r])
                accT[_COFFS[i] + r, pl.ds(g * 16, 16)] = vals

    pltpu.async_copy(accT, out_hbm.at[:, pl.ds(base, bpw)], sem_out).wait()


def _tc_body(nb, d_out, et_ref, xrow_ref, xr_ref, w_ref, b_ref, g_ref, be_ref, o_ref):
    xr = xr_ref[...]
    mean = jnp.sum(xr) * (1.0 / nb)
    var = jnp.sum((xr - mean) ** 2) * (1.0 / nb)
    a = g_ref[0, 0] * jax.lax.rsqrt(var + 1e-5)
    c = be_ref[0, 0] - mean * a

    cnT = xrow_ref[...] * a + c
    ih = lax.broadcasted_iota(jnp.int32, et_ref.shape, 0)
    eh = jnp.where(ih == _TOTC, cnT,
                   jnp.where(ih > _TOTC, 0.0, et_ref[...]))
    wfull = jnp.concatenate(
        [w_ref[...], jnp.zeros((d_out, _CPAD - _TOTC - 1), jnp.float32)], axis=1)
    z = jax.lax.dot_general(
        eh, wfull, (((0,), (1,)), ((), ())),
        preferred_element_type=jnp.float32, precision=jax.lax.Precision.DEFAULT)
    z = z + b_ref[...]
    z = z - jnp.max(z, axis=1, keepdims=True)
    ez = jnp.exp(z)
    o_ref[...] = ez / jnp.sum(ez, axis=1, keepdims=True)


def kernel(x, emb0, emb1, emb2, emb3, emb4, emb5, emb6, emb7, emb8, W, b, gamma, beta):
    tables = [emb0, emb1, emb2, emb3, emb4, emb5, emb6, emb7, emb8]
    B = x.shape[0]
    d_out = W.shape[0]
    bpw = B // _NW

    tflat = jnp.concatenate([t.reshape(-1) for t in tables]
                            + [jnp.zeros((_FPAD - _FLAT,), jnp.float32)])
    cat_t = x[:, 1:].astype(jnp.int32).T.reshape(-1)

    mesh = plsc.VectorSubcoreMesh(core_axis_name="c", subcore_axis_name="s")
    et = pl.kernel(
        functools.partial(_sc_body, bpw, B),
        out_type=jax.ShapeDtypeStruct((_CPAD, B), jnp.float32),
        mesh=mesh,
        scratch_types=[
            pltpu.VMEM((_NT * bpw,), jnp.int32),
            pltpu.VMEM((_FPAD,), jnp.float32),
            pltpu.VMEM((_CPAD, bpw), jnp.float32),
            pltpu.SemaphoreType.DMA,
            pltpu.SemaphoreType.DMA,
        ],
        compiler_params=pltpu.CompilerParams(needs_layout_passes=False),
    )(tflat, cat_t)

    xc = x[:, 0]
    xr = xc.reshape(128, B // 128)
    xrow = xc.reshape(1, B)
    b2 = b.reshape(1, d_out)
    g2 = gamma.reshape(1, 1)
    be2 = beta.reshape(1, 1)

    bb = 4096
    out = pl.pallas_call(
        functools.partial(_tc_body, float(B), d_out),
        grid=(B // bb,),
        in_specs=[
            pl.BlockSpec((_CPAD, bb), lambda i: (0, i)),
            pl.BlockSpec((1, bb), lambda i: (0, i)),
            pl.BlockSpec(xr.shape, lambda i: (0, 0)),
            pl.BlockSpec(W.shape, lambda i: (0, 0)),
            pl.BlockSpec((1, d_out), lambda i: (0, 0)),
            pl.BlockSpec((1, 1), lambda i: (0, 0)),
            pl.BlockSpec((1, 1), lambda i: (0, 0)),
        ],
        out_specs=pl.BlockSpec((bb, d_out), lambda i: (i, 0)),
        out_shape=jax.ShapeDtypeStruct((B, d_out), jnp.float32),
    )(et, xrow, xr, W, b2, g2, be2)
    return out
